# Initial kernel scaffold; baseline (speedup 1.0000x reference)
import jax
import jax.numpy as jnp
from jax.experimental import pallas as pl


def kernel(token_types, token_values, node_indices, token_table, node_table, value_W, value_b):
    raise NotImplementedError("write your pallas kernel here")


# speedup vs baseline: 6.3853x; 6.3853x over previous
import functools

import jax
import jax.numpy as jnp
from jax import lax
from jax.experimental import pallas as pl
from jax.experimental.pallas import tpu as pltpu
from jax.experimental.pallas import tpu_sc as plsc

D = 64
N = 819200
NC, NS, L = 2, 16, 16
NW = NC * NS
PER_W = N // NW
CH = 512
NG = CH // 128
NCHUNK = PER_W // CH

_mesh = plsc.VectorSubcoreMesh(
    core_axis_name="c", subcore_axis_name="s", num_cores=NC, num_subcores=NS
)


@functools.partial(
    pl.kernel,
    out_type=jax.ShapeDtypeStruct((N, D), jnp.float32),
    mesh=_mesh,
    scratch_types=[
        pltpu.VMEM((CH,), jnp.int32),
        pltpu.VMEM((CH,), jnp.int32),
        pltpu.VMEM((CH,), jnp.float32),
        pltpu.VMEM((NG, 128), jnp.int32),
        pltpu.VMEM((CH, D), jnp.float32),
        pltpu.VMEM((D,), jnp.float32),
        pltpu.SemaphoreType.DMA,
    ],
    compiler_params=pltpu.CompilerParams(use_tc_tiling_on_sc=False),
)
def _encode(types_h, nodes_h, vals_h, table_h, w_h, out_h,
            types_v, nodes_v, vals_v, idx_v, rows_v, w_v, sem):
    cid = lax.axis_index("c")
    sid = lax.axis_index("s")
    wid = sid * NC + cid
    base = wid * PER_W
    pltpu.sync_copy(w_h, w_v)

    def chunk(ci, carry):
        off = base + ci * CH
        pltpu.sync_copy(types_h.at[pl.ds(off, CH)], types_v)
        pltpu.sync_copy(nodes_h.at[pl.ds(off, CH)], nodes_v)
        pltpu.sync_copy(vals_h.at[pl.ds(off, CH)], vals_v)

        for g in range(NG):
            def fuse(k, _, g=g):
                sl = pl.ds(g * 128 + k * L, L)
                idx_v[g, pl.ds(k * L, L)] = types_v[sl] * 100 + nodes_v[sl]
                return 0
            lax.fori_loop(0, 128 // L, fuse, 0)

        descs = [
            pltpu.async_copy(
                table_h.at[idx_v.at[g]], rows_v.at[pl.ds(g * 128, 128)], sem
            )
            for g in range(NG)
        ]
        for dsc in descs:
            dsc.wait()

        def grpfn(g16, _):
            i0 = g16 * L
            vv = vals_v[pl.ds(i0, L)]
            wjs = [w_v[pl.ds(j * L, L)] for j in range(D // L)]
            for k in range(L):
                v = vv[k]
                for j in range(D // L):
                    plsc.addupdate(rows_v.at[i0 + k, pl.ds(j * L, L)], v * wjs[j])
            return 0
        lax.fori_loop(0, CH // L, grpfn, 0)

        pltpu.sync_copy(rows_v, out_h.at[pl.ds(off, CH)])
        return carry

    lax.fori_loop(0, NCHUNK, chunk, 0)


def kernel(token_types, token_values, node_indices, token_table, node_table,
           value_W, value_b):
    table = (token_table[:, None, :] + node_table[None, :, :]
             + value_b[None, None, :]).reshape(500, D)
    vals = token_values[:, 0]
    w = value_W[:, 0]
    return _encode(token_types.astype(jnp.int32), node_indices.astype(jnp.int32),
                   vals, table, w)
